# baseline (speedup 1.0000x reference)
import functools

import jax
import jax.numpy as jnp
from jax import lax
from jax.experimental import pallas as pl
from jax.experimental.pallas import tpu as pltpu
from jax.experimental.pallas import tpu_sc as plsc

N = 10000
NPAD = 10240
E = 320000
IN_CH = 128
HID = 64
OUT = 16
NREL = 2

NC = 2
NS = 16
NW = NC * NS
EPW = E // NW
CH = 80
NCHUNK = EPW // CH
G = CH // 16
TBL = NREL * NPAD
RPT = TBL // NS
CROWS = TBL // 16

BN = 512
GRID = NPAD // BN

_mesh = plsc.VectorSubcoreMesh(
    core_axis_name="c", subcore_axis_name="s", num_cores=NC, num_subcores=NS)


NBUF = 5
NOUT = NCHUNK // NBUF

_sc1_scratch = (
    [pltpu.VMEM((CH,), jnp.int32) for _ in range(NBUF)]
    + [pltpu.VMEM((CH,), jnp.int32) for _ in range(NBUF)]
    + [pltpu.VMEM((CH,), jnp.int32) for _ in range(NBUF)]
    + [pltpu.VMEM((CH,), jnp.int32) for _ in range(NBUF)]
    + [pltpu.VMEM((CH,), jnp.int32) for _ in range(NBUF)]
    + [pltpu.VMEM((CH, HID), jnp.float32) for _ in range(NBUF)]
    + [pltpu.VMEM((CROWS, 16), jnp.float32),
       pltpu.VMEM_SHARED((TBL, HID), jnp.float32)]
    + [pltpu.SemaphoreType.DMA for _ in range(3 * NBUF + 1)]
)


@functools.partial(
    pl.kernel,
    out_type=(
        jax.ShapeDtypeStruct((NC, TBL, HID), jnp.float32),
        jax.ShapeDtypeStruct((NW, CROWS, 16), jnp.float32),
        jax.ShapeDtypeStruct((E,), jnp.int32),
        jax.ShapeDtypeStruct((E,), jnp.int32),
    ),
    mesh=_mesh,
    scratch_types=_sc1_scratch,
    compiler_params=pltpu.CompilerParams(needs_layout_passes=False,
                                         use_tc_tiling_on_sc=False),
)
def _sc_layer1(ei_h, et_h, h1_h, z64_h, z16_h,
               accp_h, cntp_h, gidx_h, sidx_h, *scr):
  srcv = scr[0:NBUF]
  dstv = scr[NBUF:2 * NBUF]
  etv = scr[2 * NBUF:3 * NBUF]
  gidxv = scr[3 * NBUF:4 * NBUF]
  sidxv = scr[4 * NBUF:5 * NBUF]
  rowsv = scr[5 * NBUF:6 * NBUF]
  cntv = scr[6 * NBUF]
  acc_sh = scr[6 * NBUF + 1]
  lsem = scr[6 * NBUF + 2:6 * NBUF + 2 + NBUF]
  gsem = scr[6 * NBUF + 2 + NBUF:6 * NBUF + 2 + 2 * NBUF]
  ssem = scr[6 * NBUF + 2 + 2 * NBUF:6 * NBUF + 2 + 3 * NBUF]
  wsem = scr[6 * NBUF + 2 + 3 * NBUF]

  cid = lax.axis_index("c")
  sid = lax.axis_index("s")
  wid = cid * NS + sid
  ones16 = jnp.ones((16,), jnp.float32)

  zld0 = pltpu.async_copy(z16_h, cntv, lsem[0])
  zld1 = pltpu.async_copy(z64_h, rowsv[0], lsem[1])
  zld1.wait()
  off0 = sid * RPT
  zws = []
  for k in range(RPT // CH):
    zws.append(pltpu.async_copy(rowsv[0], acc_sh.at[pl.ds(off0 + k * CH, CH)],
                                wsem))
  zld0.wait()
  for d in zws:
    d.wait()
  plsc.subcore_barrier()

  base = wid * EPW

  def outer(ko, _):
    eo0 = pl.multiple_of(base + ko * (NBUF * CH), CH)
    lds = []
    for b in range(NBUF):
      eo = eo0 + b * CH
      lds.append((
          pltpu.async_copy(ei_h.at[0, pl.ds(eo, CH)], srcv[b], lsem[b]),
          pltpu.async_copy(ei_h.at[1, pl.ds(eo, CH)], dstv[b], lsem[b]),
          pltpu.async_copy(et_h.at[pl.ds(eo, CH)], etv[b], lsem[b]),
      ))
    gds = []
    for b in range(NBUF):
      for d in lds[b]:
        d.wait()
      eo = eo0 + b * CH

      @pl.when(ko > 0)
      def _(b=b, eo=eo):
        pltpu.make_async_copy(gidxv[b], gidx_h.at[pl.ds(eo, CH)], wsem).wait()
        pltpu.make_async_copy(sidxv[b], sidx_h.at[pl.ds(eo, CH)], wsem).wait()
        pltpu.make_async_copy(rowsv[b], acc_sh.at[sidxv[b]], ssem[b]).wait()

      for j in range(G):
        s = srcv[b][pl.ds(j * 16, 16)]
        d_ = dstv[b][pl.ds(j * 16, 16)]
        t = etv[b][pl.ds(j * 16, 16)]
        gi = t * NPAD + s
        si = t * NPAD + d_
        gidxv[b][pl.ds(j * 16, 16)] = gi
        sidxv[b][pl.ds(j * 16, 16)] = si
        row = lax.shift_right_logical(si, 4)
        col = lax.bitwise_and(si, 15)
        plsc.addupdate_scatter(cntv, [row, col], ones16)
      pltpu.async_copy(gidxv[b], gidx_h.at[pl.ds(eo, CH)], wsem)
      pltpu.async_copy(sidxv[b], sidx_h.at[pl.ds(eo, CH)], wsem)
      gds.append(pltpu.async_copy(h1_h.at[gidxv[b]], rowsv[b], gsem[b]))
    for b in range(NBUF):
      gds[b].wait()
      pltpu.async_copy(rowsv[b], acc_sh.at[sidxv[b]], ssem[b], add=True)
    return 0
  lax.fori_loop(0, NOUT, outer, 0)
  for b in range(NBUF):
    pltpu.make_async_copy(gidxv[b], gidx_h.at[pl.ds(0, CH)], wsem).wait()
    pltpu.make_async_copy(sidxv[b], sidx_h.at[pl.ds(0, CH)], wsem).wait()
    pltpu.make_async_copy(rowsv[b], acc_sh.at[sidxv[b]], ssem[b]).wait()

  plsc.subcore_barrier()
  cwr = pltpu.async_copy(cntv, cntp_h.at[wid], wsem)
  dwr = [None] * NBUF
  for k in range(RPT // CH):
    b = k % NBUF
    if dwr[b] is not None:
      dwr[b].wait()
    pltpu.async_copy(acc_sh.at[pl.ds(off0 + k * CH, CH)], rowsv[b],
                     gsem[b]).wait()
    dwr[b] = pltpu.async_copy(rowsv[b], accp_h.at[cid, pl.ds(off0 + k * CH, CH)],
                              ssem[b])
  for b in range(NBUF):
    if dwr[b] is not None:
      dwr[b].wait()
  cwr.wait()


_sc2_scratch = (
    [pltpu.VMEM((CH,), jnp.int32) for _ in range(NBUF)]
    + [pltpu.VMEM((CH,), jnp.int32) for _ in range(NBUF)]
    + [pltpu.VMEM((CH, OUT), jnp.float32) for _ in range(NBUF)]
    + [pltpu.VMEM_SHARED((TBL, OUT), jnp.float32)]
    + [pltpu.SemaphoreType.DMA for _ in range(3 * NBUF)]
)


@functools.partial(
    pl.kernel,
    out_type=jax.ShapeDtypeStruct((NC, TBL, OUT), jnp.float32),
    mesh=_mesh,
    scratch_types=_sc2_scratch,
    compiler_params=pltpu.CompilerParams(needs_layout_passes=False,
                                         use_tc_tiling_on_sc=False),
)
def _sc_layer2(gidx_h, sidx_h, h2_h, z16_h, accp_h, *scr):
  gidxv = scr[0:NBUF]
  sidxv = scr[NBUF:2 * NBUF]
  rowsv = scr[2 * NBUF:3 * NBUF]
  acc_sh = scr[3 * NBUF]
  lsem = scr[3 * NBUF + 1:3 * NBUF + 1 + NBUF]
  gsem = scr[3 * NBUF + 1 + NBUF:3 * NBUF + 1 + 2 * NBUF]
  ssem = scr[3 * NBUF + 1 + 2 * NBUF:3 * NBUF + 1 + 3 * NBUF]

  cid = lax.axis_index("c")
  sid = lax.axis_index("s")
  wid = cid * NS + sid

  pltpu.async_copy(z16_h.at[pl.ds(0, CH)], rowsv[0], lsem[0]).wait()
  off0 = sid * RPT
  zws = []
  for k in range(RPT // CH):
    zws.append(pltpu.async_copy(rowsv[0], acc_sh.at[pl.ds(off0 + k * CH, CH)],
                                ssem[0]))
  for d in zws:
    d.wait()
  plsc.subcore_barrier()

  base = wid * EPW

  def outer(ko, _):
    eo0 = pl.multiple_of(base + ko * (NBUF * CH), CH)
    lds = []
    for b in range(NBUF):
      eo = eo0 + b * CH

      @pl.when(ko > 0)
      def _(b=b):
        pltpu.make_async_copy(rowsv[b], acc_sh.at[sidxv[b]], ssem[b]).wait()

      lds.append((
          pltpu.async_copy(gidx_h.at[pl.ds(eo, CH)], gidxv[b], lsem[b]),
          pltpu.async_copy(sidx_h.at[pl.ds(eo, CH)], sidxv[b], lsem[b]),
      ))
    gds = []
    for b in range(NBUF):
      for d in lds[b]:
        d.wait()
      gds.append(pltpu.async_copy(h2_h.at[gidxv[b]], rowsv[b], gsem[b]))
    for b in range(NBUF):
      gds[b].wait()
      pltpu.async_copy(rowsv[b], acc_sh.at[sidxv[b]], ssem[b], add=True)
    return 0
  lax.fori_loop(0, NOUT, outer, 0)
  for b in range(NBUF):
    pltpu.make_async_copy(rowsv[b], acc_sh.at[sidxv[b]], ssem[b]).wait()

  plsc.subcore_barrier()
  dwr = [None] * NBUF
  for k in range(RPT // CH):
    b = k % NBUF
    if dwr[b] is not None:
      dwr[b].wait()
    pltpu.async_copy(acc_sh.at[pl.ds(off0 + k * CH, CH)], rowsv[b],
                     gsem[b]).wait()
    dwr[b] = pltpu.async_copy(rowsv[b], accp_h.at[cid, pl.ds(off0 + k * CH, CH)],
                              ssem[b])
  for b in range(NBUF):
    if dwr[b] is not None:
      dwr[b].wait()


def _tc_dense1_body(x_ref, root_ref, rel_ref, b_ref, r1_ref, h_ref):
  xb = x_ref[...]
  r1_ref[...] = jnp.dot(xb, root_ref[...],
                        preferred_element_type=jnp.float32) + b_ref[...]
  h0 = jnp.dot(xb, rel_ref[0], preferred_element_type=jnp.float32)
  h1 = jnp.dot(xb, rel_ref[1], preferred_element_type=jnp.float32)
  h_ref[...] = jnp.stack([h0, h1])


def _tc_dense1(xp, root1, rel1, b1):
  return pl.pallas_call(
      _tc_dense1_body,
      grid=(GRID,),
      in_specs=[
          pl.BlockSpec((BN, IN_CH), lambda i: (i, 0)),
          pl.BlockSpec((IN_CH, HID), lambda i: (0, 0)),
          pl.BlockSpec((NREL, IN_CH, HID), lambda i: (0, 0, 0)),
          pl.BlockSpec((1, HID), lambda i: (0, 0)),
      ],
      out_specs=[
          pl.BlockSpec((BN, HID), lambda i: (i, 0)),
          pl.BlockSpec((NREL, BN, HID), lambda i: (0, i, 0)),
      ],
      out_shape=[
          jax.ShapeDtypeStruct((NPAD, HID), jnp.float32),
          jax.ShapeDtypeStruct((NREL, NPAD, HID), jnp.float32),
      ],
  )(xp, root1, rel1, b1)


CPR = TBL // 128


def _tc_inv_body(cnt_ref, inv_ref):
  s = jnp.sum(cnt_ref[...], axis=0)
  inv_ref[...] = 1.0 / jnp.maximum(s, 1.0)


def _tc_inv(cntpx):
  return pl.pallas_call(
      _tc_inv_body,
      grid=(1,),
      in_specs=[pl.BlockSpec((NW, CPR, 128), lambda i: (0, 0, 0))],
      out_specs=pl.BlockSpec((CPR, 128), lambda i: (0, 0)),
      out_shape=jax.ShapeDtypeStruct((CPR, 128), jnp.float32),
  )(cntpx)


def _tc_combine1_body(r1_ref, acc0_ref, acc1_ref, inv_ref,
                      root2_ref, rel2_ref, b2_ref,
                      r2_ref, h2_ref):
  inv = inv_ref[...]
  a0b = acc0_ref[...]
  a1b = acc1_ref[...]
  a0 = a0b[0] + a0b[1]
  a1 = a1b[0] + a1b[1]
  out1 = r1_ref[...] + a0 * inv[0][:, None] + a1 * inv[1][:, None]
  out1 = jnp.maximum(out1, 0.0)
  r2_ref[...] = jnp.dot(out1, root2_ref[...],
                        preferred_element_type=jnp.float32) + b2_ref[...]
  h0 = jnp.dot(out1, rel2_ref[0], preferred_element_type=jnp.float32)
  h1 = jnp.dot(out1, rel2_ref[1], preferred_element_type=jnp.float32)
  h2_ref[...] = jnp.stack([h0, h1])


def _tc_combine1(r1, accp, inv, root2, rel2, b2):
  return pl.pallas_call(
      _tc_combine1_body,
      grid=(GRID,),
      in_specs=[
          pl.BlockSpec((BN, HID), lambda i: (i, 0)),
          pl.BlockSpec((NC, BN, HID), lambda i: (0, i, 0)),
          pl.BlockSpec((NC, BN, HID), lambda i: (0, GRID + i, 0)),
          pl.BlockSpec((NREL, BN), lambda i: (0, i)),
          pl.BlockSpec((HID, OUT), lambda i: (0, 0)),
          pl.BlockSpec((NREL, HID, OUT), lambda i: (0, 0, 0)),
          pl.BlockSpec((1, OUT), lambda i: (0, 0)),
      ],
      out_specs=[
          pl.BlockSpec((BN, OUT), lambda i: (i, 0)),
          pl.BlockSpec((NREL, BN, OUT), lambda i: (0, i, 0)),
      ],
      out_shape=[
          jax.ShapeDtypeStruct((NPAD, OUT), jnp.float32),
          jax.ShapeDtypeStruct((NREL, NPAD, OUT), jnp.float32),
      ],
  )(r1, accp, accp, inv, root2, rel2, b2)


PR = NPAD * OUT // 128
BNE = 128
GRIDE = PR // BNE


def _tc_final_body(r2_ref, acc0_ref, acc1_ref, inv0_ref, inv1_ref, out_ref):
  a0b = acc0_ref[...]
  a1b = acc1_ref[...]
  a0 = a0b[0] + a0b[1]
  a1 = a1b[0] + a1b[1]
  out_ref[...] = r2_ref[...] + a0 * inv0_ref[...] + a1 * inv1_ref[...]


def _tc_final(r2x, accp2x, inv0x, inv1x):
  return pl.pallas_call(
      _tc_final_body,
      grid=(GRIDE,),
      in_specs=[
          pl.BlockSpec((BNE, 128), lambda i: (i, 0)),
          pl.BlockSpec((NC, BNE, 128), lambda i: (0, i, 0)),
          pl.BlockSpec((NC, BNE, 128), lambda i: (0, GRIDE + i, 0)),
          pl.BlockSpec((BNE, 128), lambda i: (i, 0)),
          pl.BlockSpec((BNE, 128), lambda i: (i, 0)),
      ],
      out_specs=pl.BlockSpec((BNE, 128), lambda i: (i, 0)),
      out_shape=jax.ShapeDtypeStruct((PR, 128), jnp.float32),
  )(r2x, accp2x, accp2x, inv0x, inv1x)


@jax.jit
def kernel(x, edge_index, edge_type, root1, rel1, b1, root2, rel2, b2):
  z64 = jnp.zeros((CH, HID), jnp.float32)
  z16 = jnp.zeros((CROWS, 16), jnp.float32)

  r1, h1 = _tc_dense1(x, root1, rel1, b1.reshape(1, HID))
  accp, cntp, gidx, sidx = _sc_layer1(
      edge_index, edge_type, h1.reshape(TBL, HID), z64, z16)
  inv = _tc_inv(cntp.reshape(NW, CPR, 128)).reshape(NREL, NPAD)
  r2, h2 = _tc_combine1(r1, accp, inv, root2, rel2, b2.reshape(1, OUT))
  accp2 = _sc_layer2(gidx, sidx, h2.reshape(TBL, OUT), z16)
  invx = jnp.broadcast_to(inv[:, :, None], (NREL, NPAD, OUT))
  out = _tc_final(
      r2.reshape(PR, 128),
      accp2.reshape(NC, 2 * PR, 128),
      invx[0].reshape(PR, 128),
      invx[1].reshape(PR, 128))
  return out.reshape(NPAD, OUT)[:N]


# speedup vs baseline: 1.0979x; 1.0979x over previous
import functools

import jax
import jax.numpy as jnp
from jax import lax
from jax.experimental import pallas as pl
from jax.experimental.pallas import tpu as pltpu
from jax.experimental.pallas import tpu_sc as plsc

N = 10000
NPAD = 10240
E = 320000
IN_CH = 128
HID = 64
OUT = 16
NREL = 2

NC = 2
NS = 16
NW = NC * NS
EPW = E // NW
CH = 80
NCHUNK = EPW // CH
G = CH // 16
TBL = NREL * NPAD
RPT = TBL // NS
CROWS = TBL // 16

BN = 512
GRID = NPAD // BN

_mesh = plsc.VectorSubcoreMesh(
    core_axis_name="c", subcore_axis_name="s", num_cores=NC, num_subcores=NS)


NBUF = 5
NOUT = NCHUNK // NBUF

_prep_scratch = (
    [pltpu.VMEM((CH,), jnp.int32) for _ in range(NBUF)]
    + [pltpu.VMEM((CH,), jnp.int32) for _ in range(NBUF)]
    + [pltpu.VMEM((CH,), jnp.int32) for _ in range(NBUF)]
    + [pltpu.VMEM((CH,), jnp.int32) for _ in range(NBUF)]
    + [pltpu.VMEM((CH,), jnp.int32) for _ in range(NBUF)]
    + [pltpu.VMEM((CROWS, 16), jnp.float32)]
    + [pltpu.SemaphoreType.DMA for _ in range(NBUF + 1)]
)


@functools.partial(
    pl.kernel,
    out_type=(
        jax.ShapeDtypeStruct((NW, CROWS, 16), jnp.float32),
        jax.ShapeDtypeStruct((E,), jnp.int32),
        jax.ShapeDtypeStruct((E,), jnp.int32),
    ),
    mesh=_mesh,
    scratch_types=_prep_scratch,
    compiler_params=pltpu.CompilerParams(needs_layout_passes=False,
                                         use_tc_tiling_on_sc=False),
)
def _sc_prep(ei_h, et_h, z16_h, cntp_h, gidx_h, sidx_h, *scr):
  srcv = scr[0:NBUF]
  dstv = scr[NBUF:2 * NBUF]
  etv = scr[2 * NBUF:3 * NBUF]
  gidxv = scr[3 * NBUF:4 * NBUF]
  sidxv = scr[4 * NBUF:5 * NBUF]
  cntv = scr[5 * NBUF]
  lsem = scr[5 * NBUF + 1:5 * NBUF + 1 + NBUF]
  wsem = scr[5 * NBUF + 1 + NBUF]

  cid = lax.axis_index("c")
  sid = lax.axis_index("s")
  wid = cid * NS + sid
  ones16 = jnp.ones((16,), jnp.float32)

  pltpu.async_copy(z16_h, cntv, lsem[0]).wait()
  base = wid * EPW

  def outer(ko, _):
    eo0 = pl.multiple_of(base + ko * (NBUF * CH), CH)
    lds = []
    for b in range(NBUF):
      eo = eo0 + b * CH
      lds.append((
          pltpu.async_copy(ei_h.at[0, pl.ds(eo, CH)], srcv[b], lsem[b]),
          pltpu.async_copy(ei_h.at[1, pl.ds(eo, CH)], dstv[b], lsem[b]),
          pltpu.async_copy(et_h.at[pl.ds(eo, CH)], etv[b], lsem[b]),
      ))
    for b in range(NBUF):
      for d in lds[b]:
        d.wait()
      eo = eo0 + b * CH

      @pl.when(ko > 0)
      def _(b=b, eo=eo):
        pltpu.make_async_copy(gidxv[b], gidx_h.at[pl.ds(eo, CH)], wsem).wait()
        pltpu.make_async_copy(sidxv[b], sidx_h.at[pl.ds(eo, CH)], wsem).wait()

      for j in range(G):
        s = srcv[b][pl.ds(j * 16, 16)]
        d_ = dstv[b][pl.ds(j * 16, 16)]
        t = etv[b][pl.ds(j * 16, 16)]
        gi = t * NPAD + s
        si = t * NPAD + d_
        gidxv[b][pl.ds(j * 16, 16)] = gi
        sidxv[b][pl.ds(j * 16, 16)] = si
        row = lax.shift_right_logical(si, 4)
        col = lax.bitwise_and(si, 15)
        plsc.addupdate_scatter(cntv, [row, col], ones16)
      pltpu.async_copy(gidxv[b], gidx_h.at[pl.ds(eo, CH)], wsem)
      pltpu.async_copy(sidxv[b], sidx_h.at[pl.ds(eo, CH)], wsem)
    return 0
  lax.fori_loop(0, NOUT, outer, 0)
  for b in range(NBUF):
    pltpu.make_async_copy(gidxv[b], gidx_h.at[pl.ds(0, CH)], wsem).wait()
    pltpu.make_async_copy(sidxv[b], sidx_h.at[pl.ds(0, CH)], wsem).wait()
  pltpu.sync_copy(cntv, cntp_h.at[wid])


def _make_sc_gather(width):
  scratch = (
      [pltpu.VMEM((CH,), jnp.int32) for _ in range(NBUF)]
      + [pltpu.VMEM((CH,), jnp.int32) for _ in range(NBUF)]
      + [pltpu.VMEM((CH, width), jnp.float32) for _ in range(NBUF)]
      + [pltpu.VMEM_SHARED((TBL, width), jnp.float32)]
      + [pltpu.SemaphoreType.DMA for _ in range(3 * NBUF)]
  )

  @functools.partial(
      pl.kernel,
      out_type=jax.ShapeDtypeStruct((NC, TBL, width), jnp.float32),
      mesh=_mesh,
      scratch_types=scratch,
      compiler_params=pltpu.CompilerParams(needs_layout_passes=False,
                                           use_tc_tiling_on_sc=False),
  )
  def gather_kernel(gidx_h, sidx_h, tbl_h, z_h, accp_h, *scr):
    gidxv = scr[0:NBUF]
    sidxv = scr[NBUF:2 * NBUF]
    rowsv = scr[2 * NBUF:3 * NBUF]
    acc_sh = scr[3 * NBUF]
    lsem = scr[3 * NBUF + 1:3 * NBUF + 1 + NBUF]
    gsem = scr[3 * NBUF + 1 + NBUF:3 * NBUF + 1 + 2 * NBUF]
    ssem = scr[3 * NBUF + 1 + 2 * NBUF:3 * NBUF + 1 + 3 * NBUF]

    cid = lax.axis_index("c")
    sid = lax.axis_index("s")
    wid = cid * NS + sid

    pltpu.async_copy(z_h, rowsv[0], lsem[0]).wait()
    off0 = sid * RPT
    zws = []
    for k in range(RPT // CH):
      zws.append(pltpu.async_copy(rowsv[0],
                                  acc_sh.at[pl.ds(off0 + k * CH, CH)],
                                  ssem[0]))
    for d in zws:
      d.wait()
    plsc.subcore_barrier()

    base = wid * EPW

    def outer(ko, _):
      eo0 = pl.multiple_of(base + ko * (NBUF * CH), CH)
      lds = []
      for b in range(NBUF):
        eo = eo0 + b * CH

        @pl.when(ko > 0)
        def _(b=b):
          pltpu.make_async_copy(rowsv[b], acc_sh.at[sidxv[b]], ssem[b]).wait()

        lds.append((
            pltpu.async_copy(gidx_h.at[pl.ds(eo, CH)], gidxv[b], lsem[b]),
            pltpu.async_copy(sidx_h.at[pl.ds(eo, CH)], sidxv[b], lsem[b]),
        ))
      gds = []
      for b in range(NBUF):
        for d in lds[b]:
          d.wait()
        gds.append(pltpu.async_copy(tbl_h.at[gidxv[b]], rowsv[b], gsem[b]))
      for b in range(NBUF):
        gds[b].wait()
        pltpu.async_copy(rowsv[b], acc_sh.at[sidxv[b]], ssem[b], add=True)
      return 0
    lax.fori_loop(0, NOUT, outer, 0)
    for b in range(NBUF):
      pltpu.make_async_copy(rowsv[b], acc_sh.at[sidxv[b]], ssem[b]).wait()

    plsc.subcore_barrier()
    dwr = [None] * NBUF
    for k in range(RPT // CH):
      b = k % NBUF
      if dwr[b] is not None:
        dwr[b].wait()
      pltpu.async_copy(acc_sh.at[pl.ds(off0 + k * CH, CH)], rowsv[b],
                       gsem[b]).wait()
      dwr[b] = pltpu.async_copy(rowsv[b],
                                accp_h.at[cid, pl.ds(off0 + k * CH, CH)],
                                ssem[b])
    for b in range(NBUF):
      if dwr[b] is not None:
        dwr[b].wait()

  return gather_kernel


_sc_gather1 = _make_sc_gather(HID)
_sc_gather2 = _make_sc_gather(OUT)


def _tc_dense1_body(x_ref, root_ref, rel_ref, b_ref, r1_ref, h_ref):
  xb = x_ref[...]
  r1_ref[...] = jnp.dot(xb, root_ref[...],
                        preferred_element_type=jnp.float32) + b_ref[...]
  h0 = jnp.dot(xb, rel_ref[0], preferred_element_type=jnp.float32)
  h1 = jnp.dot(xb, rel_ref[1], preferred_element_type=jnp.float32)
  h_ref[...] = jnp.stack([h0, h1])


def _tc_dense1(xp, root1, rel1, b1):
  return pl.pallas_call(
      _tc_dense1_body,
      grid=(GRID,),
      in_specs=[
          pl.BlockSpec((BN, IN_CH), lambda i: (i, 0)),
          pl.BlockSpec((IN_CH, HID), lambda i: (0, 0)),
          pl.BlockSpec((NREL, IN_CH, HID), lambda i: (0, 0, 0)),
          pl.BlockSpec((1, HID), lambda i: (0, 0)),
      ],
      out_specs=[
          pl.BlockSpec((BN, HID), lambda i: (i, 0)),
          pl.BlockSpec((NREL, BN, HID), lambda i: (0, i, 0)),
      ],
      out_shape=[
          jax.ShapeDtypeStruct((NPAD, HID), jnp.float32),
          jax.ShapeDtypeStruct((NREL, NPAD, HID), jnp.float32),
      ],
  )(xp, root1, rel1, b1)


CPR = TBL // 128


def _tc_inv_body(cnt_ref, inv_ref):
  s = jnp.sum(cnt_ref[...], axis=0)
  inv_ref[...] = 1.0 / jnp.maximum(s, 1.0)


def _tc_inv(cntpx):
  return pl.pallas_call(
      _tc_inv_body,
      grid=(1,),
      in_specs=[pl.BlockSpec((NW, CPR, 128), lambda i: (0, 0, 0))],
      out_specs=pl.BlockSpec((CPR, 128), lambda i: (0, 0)),
      out_shape=jax.ShapeDtypeStruct((CPR, 128), jnp.float32),
  )(cntpx)


def _tc_combine1_body(r1_ref, acc0_ref, acc1_ref, inv_ref,
                      root2_ref, rel2_ref, b2_ref,
                      r2_ref, h2_ref):
  inv = inv_ref[...]
  a0b = acc0_ref[...]
  a1b = acc1_ref[...]
  a0 = a0b[0] + a0b[1]
  a1 = a1b[0] + a1b[1]
  out1 = r1_ref[...] + a0 * inv[0][:, None] + a1 * inv[1][:, None]
  out1 = jnp.maximum(out1, 0.0)
  r2_ref[...] = jnp.dot(out1, root2_ref[...],
                        preferred_element_type=jnp.float32) + b2_ref[...]
  h0 = jnp.dot(out1, rel2_ref[0], preferred_element_type=jnp.float32)
  h1 = jnp.dot(out1, rel2_ref[1], preferred_element_type=jnp.float32)
  h2_ref[...] = jnp.stack([h0, h1])


def _tc_combine1(r1, accp, inv, root2, rel2, b2):
  return pl.pallas_call(
      _tc_combine1_body,
      grid=(GRID,),
      in_specs=[
          pl.BlockSpec((BN, HID), lambda i: (i, 0)),
          pl.BlockSpec((NC, BN, HID), lambda i: (0, i, 0)),
          pl.BlockSpec((NC, BN, HID), lambda i: (0, GRID + i, 0)),
          pl.BlockSpec((NREL, BN), lambda i: (0, i)),
          pl.BlockSpec((HID, OUT), lambda i: (0, 0)),
          pl.BlockSpec((NREL, HID, OUT), lambda i: (0, 0, 0)),
          pl.BlockSpec((1, OUT), lambda i: (0, 0)),
      ],
      out_specs=[
          pl.BlockSpec((BN, OUT), lambda i: (i, 0)),
          pl.BlockSpec((NREL, BN, OUT), lambda i: (0, i, 0)),
      ],
      out_shape=[
          jax.ShapeDtypeStruct((NPAD, OUT), jnp.float32),
          jax.ShapeDtypeStruct((NREL, NPAD, OUT), jnp.float32),
      ],
  )(r1, accp, accp, inv, root2, rel2, b2)


PR = NPAD * OUT // 128
BNE = 128
GRIDE = PR // BNE


def _tc_final_body(r2_ref, acc0_ref, acc1_ref, inv0_ref, inv1_ref, out_ref):
  a0b = acc0_ref[...]
  a1b = acc1_ref[...]
  a0 = a0b[0] + a0b[1]
  a1 = a1b[0] + a1b[1]
  out_ref[...] = r2_ref[...] + a0 * inv0_ref[...] + a1 * inv1_ref[...]


def _tc_final(r2x, accp2x, inv0x, inv1x):
  return pl.pallas_call(
      _tc_final_body,
      grid=(GRIDE,),
      in_specs=[
          pl.BlockSpec((BNE, 128), lambda i: (i, 0)),
          pl.BlockSpec((NC, BNE, 128), lambda i: (0, i, 0)),
          pl.BlockSpec((NC, BNE, 128), lambda i: (0, GRIDE + i, 0)),
          pl.BlockSpec((BNE, 128), lambda i: (i, 0)),
          pl.BlockSpec((BNE, 128), lambda i: (i, 0)),
      ],
      out_specs=pl.BlockSpec((BNE, 128), lambda i: (i, 0)),
      out_shape=jax.ShapeDtypeStruct((PR, 128), jnp.float32),
  )(r2x, accp2x, accp2x, inv0x, inv1x)


@jax.jit
def kernel(x, edge_index, edge_type, root1, rel1, b1, root2, rel2, b2):
  z64 = jnp.zeros((CH, HID), jnp.float32)
  z16 = jnp.zeros((CROWS, 16), jnp.float32)
  zout = jnp.zeros((CH, OUT), jnp.float32)

  cntp, gidx, sidx = _sc_prep(edge_index, edge_type, z16)
  r1, h1 = _tc_dense1(x, root1, rel1, b1.reshape(1, HID))
  accp = _sc_gather1(gidx, sidx, h1.reshape(TBL, HID), z64)
  inv = _tc_inv(cntp.reshape(NW, CPR, 128)).reshape(NREL, NPAD)
  r2, h2 = _tc_combine1(r1, accp, inv, root2, rel2, b2.reshape(1, OUT))
  accp2 = _sc_gather2(gidx, sidx, h2.reshape(TBL, OUT), zout)
  invx = jnp.broadcast_to(inv[:, :, None], (NREL, NPAD, OUT))
  out = _tc_final(
      r2.reshape(PR, 128),
      accp2.reshape(NC, 2 * PR, 128),
      invx[0].reshape(PR, 128),
      invx[1].reshape(PR, 128))
  return out.reshape(NPAD, OUT)[:N]
